# Initial kernel scaffold; baseline (speedup 1.0000x reference)
import jax
import jax.numpy as jnp
from jax.experimental import pallas as pl


def kernel(x, edge_index, W2, b2, Wl, bl):
    raise NotImplementedError("write your pallas kernel here")


# speedup vs baseline: 39.8833x; 39.8833x over previous
import functools

import jax
import jax.numpy as jnp
from jax import lax
from jax.experimental import pallas as pl
from jax.experimental.pallas import tpu as pltpu
from jax.experimental.pallas import tpu_sc as plsc

N = 10000
F = 128
C = 16
NC = 2
NS = 16
NW = NC * NS
CHUNK = 128
NCH = 80
EPT = NCH * CHUNK
EPAD = NW * EPT
NPAD = 10240
RPT = NPAD // NS
BLK = 2000
GRID = N // BLK

_mesh = plsc.VectorSubcoreMesh(core_axis_name="c", subcore_axis_name="s")
_sc_params = pltpu.CompilerParams(use_tc_tiling_on_sc=False)


@functools.partial(
    pl.kernel,
    mesh=_mesh,
    out_type=jax.ShapeDtypeStruct((NC, NPAD, C), jnp.float32),
    compiler_params=_sc_params,
    scratch_types=[
        pltpu.VMEM((NCH, CHUNK), jnp.int32),
        pltpu.VMEM((CHUNK, C), jnp.float32),
        pltpu.VMEM_SHARED((NPAD, C), jnp.float32),
    ],
)
def _degree_kernel(dst_hbm, zo_hbm, out_hbm, dst_v, buf_v, acc_sh):
    cid = lax.axis_index("c")
    sid = lax.axis_index("s")
    wid = cid * NS + sid
    pltpu.sync_copy(zo_hbm.at[0], buf_v)
    for k in range(RPT // CHUNK):
        pltpu.sync_copy(buf_v, acc_sh.at[pl.ds(sid * RPT + k * CHUNK, CHUNK)])
    pltpu.sync_copy(zo_hbm.at[1], buf_v)
    pltpu.sync_copy(dst_hbm.at[wid], dst_v)
    plsc.subcore_barrier()

    def body(j, carry):
        pltpu.sync_copy(buf_v, acc_sh.at[dst_v.at[j]], add=True)
        return carry

    lax.fori_loop(0, NCH, body, 0)
    plsc.subcore_barrier()
    pltpu.sync_copy(acc_sh.at[pl.ds(sid * RPT, RPT)],
                    out_hbm.at[cid, pl.ds(sid * RPT, RPT)])


@functools.partial(
    pl.kernel,
    mesh=_mesh,
    out_type=jax.ShapeDtypeStruct((NC, NPAD, C), jnp.float32),
    compiler_params=_sc_params,
    scratch_types=[
        pltpu.VMEM((NCH, CHUNK), jnp.int32),
        pltpu.VMEM((NCH, CHUNK), jnp.int32),
        pltpu.VMEM((CHUNK, C), jnp.float32),
        pltpu.VMEM((CHUNK, C), jnp.float32),
        pltpu.VMEM_SHARED((NPAD, C), jnp.float32),
        pltpu.SemaphoreType.DMA,
        pltpu.SemaphoreType.DMA,
    ],
)
def _propagate_kernel(src_hbm, dst_hbm, g_hbm, zo_hbm, out_hbm,
                      src_v, dst_v, row0_v, row1_v, acc_sh, sem0, sem1):
    cid = lax.axis_index("c")
    sid = lax.axis_index("s")
    wid = cid * NS + sid
    pltpu.sync_copy(zo_hbm.at[0], row0_v)
    for k in range(RPT // CHUNK):
        pltpu.sync_copy(row0_v, acc_sh.at[pl.ds(sid * RPT + k * CHUNK, CHUNK)])
    pltpu.sync_copy(src_hbm.at[wid], src_v)
    pltpu.sync_copy(dst_hbm.at[wid], dst_v)
    plsc.subcore_barrier()

    pltpu.async_copy(g_hbm.at[src_v.at[0]], row0_v, sem0)

    def body(i, carry):
        j0 = i * 2
        pltpu.make_async_copy(g_hbm.at[src_v.at[j0]], row0_v, sem0).wait()
        pltpu.async_copy(g_hbm.at[src_v.at[j0 + 1]], row1_v, sem1)
        pltpu.sync_copy(row0_v, acc_sh.at[dst_v.at[j0]], add=True)
        pltpu.make_async_copy(g_hbm.at[src_v.at[j0 + 1]], row1_v, sem1).wait()
        pltpu.async_copy(g_hbm.at[src_v.at[j0 + 2]], row0_v, sem0)
        pltpu.sync_copy(row1_v, acc_sh.at[dst_v.at[j0 + 1]], add=True)
        return carry

    lax.fori_loop(0, NCH // 2 - 1, body, 0)
    j0 = NCH - 2
    pltpu.make_async_copy(g_hbm.at[src_v.at[j0]], row0_v, sem0).wait()
    pltpu.async_copy(g_hbm.at[src_v.at[j0 + 1]], row1_v, sem1)
    pltpu.sync_copy(row0_v, acc_sh.at[dst_v.at[j0]], add=True)
    pltpu.make_async_copy(g_hbm.at[src_v.at[j0 + 1]], row1_v, sem1).wait()
    pltpu.sync_copy(row1_v, acc_sh.at[dst_v.at[j0 + 1]], add=True)

    plsc.subcore_barrier()
    pltpu.sync_copy(acc_sh.at[pl.ds(sid * RPT, RPT)],
                    out_hbm.at[cid, pl.ds(sid * RPT, RPT)])


def _dense_in_body(deg_ref, x_ref, w2_ref, g_ref, dis_ref):
    d = deg_ref[0, :, 0:1] + deg_ref[1, :, 0:1] + 1.0
    dis = lax.rsqrt(d)
    h = jnp.dot(x_ref[...], w2_ref[...], preferred_element_type=jnp.float32)
    g_ref[...] = h * dis
    dis_ref[...] = dis


_dense_in = pl.pallas_call(
    _dense_in_body,
    grid=(GRID,),
    in_specs=[
        pl.BlockSpec((NC, BLK, C), lambda j: (0, j, 0)),
        pl.BlockSpec((BLK, F), lambda j: (j, 0)),
        pl.BlockSpec((F, C), lambda j: (0, 0)),
    ],
    out_specs=[
        pl.BlockSpec((BLK, C), lambda j: (j, 0)),
        pl.BlockSpec((BLK, 1), lambda j: (j, 0)),
    ],
    out_shape=[
        jax.ShapeDtypeStruct((N, C), jnp.float32),
        jax.ShapeDtypeStruct((N, 1), jnp.float32),
    ],
)


def _dense_out_body(s_ref, g_ref, dis_ref, b2_ref, wl_ref, bl_ref, o_ref):
    t = (s_ref[0] + s_ref[1] + g_ref[...]) * dis_ref[...] + b2_ref[...]
    t = jnp.maximum(t, 0.0)
    o_ref[...] = jnp.dot(t, wl_ref[...], preferred_element_type=jnp.float32) + bl_ref[...]


_dense_out = pl.pallas_call(
    _dense_out_body,
    grid=(GRID,),
    in_specs=[
        pl.BlockSpec((NC, BLK, C), lambda j: (0, j, 0)),
        pl.BlockSpec((BLK, C), lambda j: (j, 0)),
        pl.BlockSpec((BLK, 1), lambda j: (j, 0)),
        pl.BlockSpec((1, C), lambda j: (0, 0)),
        pl.BlockSpec((C, C), lambda j: (0, 0)),
        pl.BlockSpec((1, C), lambda j: (0, 0)),
    ],
    out_specs=pl.BlockSpec((BLK, C), lambda j: (j, 0)),
    out_shape=jax.ShapeDtypeStruct((N, C), jnp.float32),
)


def kernel(x, edge_index, W2, b2, Wl, bl):
    E = edge_index.shape[1]
    src = edge_index[0].astype(jnp.int32)
    dst = edge_index[1].astype(jnp.int32)
    pad = EPAD - E
    src_p = jnp.concatenate([src, jnp.zeros((pad,), jnp.int32)]).reshape(NW, NCH, CHUNK)
    dst_p = jnp.concatenate([dst, jnp.full((pad,), N, jnp.int32)]).reshape(NW, NCH, CHUNK)
    zo = jnp.stack([jnp.zeros((CHUNK, C), jnp.float32),
                    jnp.ones((CHUNK, C), jnp.float32)])
    deg2 = _degree_kernel(dst_p, zo)
    g, dis = _dense_in(deg2, x, W2)
    s2 = _propagate_kernel(src_p, dst_p, g, zo)
    return _dense_out(s2, g, dis, b2.reshape(1, C), Wl, bl.reshape(1, C))


# speedup vs baseline: 60.8004x; 1.5245x over previous
import functools

import jax
import jax.numpy as jnp
from jax import lax
from jax.experimental import pallas as pl
from jax.experimental.pallas import tpu as pltpu
from jax.experimental.pallas import tpu_sc as plsc

N = 10000
F = 128
C = 16
NC = 2
NS = 16
NW = NC * NS
CHUNK = 128
NCH = 80
EPT = NCH * CHUNK
EPAD = NW * EPT
NPAD = 10240
RPT = NPAD // NS
BLK = 2000
GRID = N // BLK

_mesh = plsc.VectorSubcoreMesh(core_axis_name="c", subcore_axis_name="s")
_sc_params = pltpu.CompilerParams(use_tc_tiling_on_sc=False)


@functools.partial(
    pl.kernel,
    mesh=_mesh,
    out_type=jax.ShapeDtypeStruct((NC, NPAD, 1), jnp.float32),
    compiler_params=_sc_params,
    scratch_types=[
        pltpu.VMEM((NCH, CHUNK), jnp.int32),
        pltpu.VMEM((CHUNK, 1), jnp.float32),
        pltpu.VMEM_SHARED((NPAD, 1), jnp.float32),
    ],
)
def _degree_kernel(dst_hbm, zo_hbm, out_hbm, dst_v, buf_v, acc_sh):
    cid = lax.axis_index("c")
    sid = lax.axis_index("s")
    wid = cid * NS + sid
    pltpu.sync_copy(zo_hbm.at[0], buf_v)
    for k in range(RPT // CHUNK):
        pltpu.sync_copy(buf_v, acc_sh.at[pl.ds(sid * RPT + k * CHUNK, CHUNK)])
    pltpu.sync_copy(zo_hbm.at[1], buf_v)
    pltpu.sync_copy(dst_hbm.at[wid], dst_v)
    plsc.subcore_barrier()

    def body(j, carry):
        pltpu.sync_copy(buf_v, acc_sh.at[dst_v.at[j]], add=True)
        return carry

    lax.fori_loop(0, NCH, body, 0)
    plsc.subcore_barrier()
    pltpu.sync_copy(acc_sh.at[pl.ds(sid * RPT, RPT)],
                    out_hbm.at[cid, pl.ds(sid * RPT, RPT)])


@functools.partial(
    pl.kernel,
    mesh=_mesh,
    out_type=jax.ShapeDtypeStruct((NC, NPAD, C), jnp.float32),
    compiler_params=_sc_params,
    scratch_types=[
        pltpu.VMEM((NCH, CHUNK), jnp.int32),
        pltpu.VMEM((NCH, CHUNK), jnp.int32),
        pltpu.VMEM((CHUNK, C), jnp.float32),
        pltpu.VMEM((CHUNK, C), jnp.float32),
        pltpu.VMEM((N // NS, C), jnp.float32),
        pltpu.VMEM_SHARED((N, C), jnp.float32),
        pltpu.VMEM_SHARED((NPAD, C), jnp.float32),
        pltpu.SemaphoreType.DMA,
        pltpu.SemaphoreType.DMA,
    ],
)
def _propagate_kernel(src_hbm, dst_hbm, g_hbm, zo_hbm, out_hbm,
                      src_v, dst_v, row0_v, row1_v, stage_v,
                      g_sh, acc_sh, sem0, sem1):
    cid = lax.axis_index("c")
    sid = lax.axis_index("s")
    wid = cid * NS + sid
    gpt = N // NS
    pltpu.sync_copy(g_hbm.at[pl.ds(sid * gpt, gpt)], stage_v)
    pltpu.sync_copy(stage_v, g_sh.at[pl.ds(sid * gpt, gpt)])
    pltpu.sync_copy(zo_hbm.at[0], row0_v)
    for k in range(RPT // CHUNK):
        pltpu.sync_copy(row0_v, acc_sh.at[pl.ds(sid * RPT + k * CHUNK, CHUNK)])
    pltpu.sync_copy(src_hbm.at[wid], src_v)
    pltpu.sync_copy(dst_hbm.at[wid], dst_v)
    plsc.subcore_barrier()

    pltpu.async_copy(g_sh.at[src_v.at[0]], row0_v, sem0)

    def body(i, carry):
        j0 = i * 2
        pltpu.make_async_copy(g_sh.at[src_v.at[j0]], row0_v, sem0).wait()
        pltpu.async_copy(g_sh.at[src_v.at[j0 + 1]], row1_v, sem1)
        pltpu.sync_copy(row0_v, acc_sh.at[dst_v.at[j0]], add=True)
        pltpu.make_async_copy(g_sh.at[src_v.at[j0 + 1]], row1_v, sem1).wait()
        pltpu.async_copy(g_sh.at[src_v.at[j0 + 2]], row0_v, sem0)
        pltpu.sync_copy(row1_v, acc_sh.at[dst_v.at[j0 + 1]], add=True)
        return carry

    lax.fori_loop(0, NCH // 2 - 1, body, 0)
    j0 = NCH - 2
    pltpu.make_async_copy(g_sh.at[src_v.at[j0]], row0_v, sem0).wait()
    pltpu.async_copy(g_sh.at[src_v.at[j0 + 1]], row1_v, sem1)
    pltpu.sync_copy(row0_v, acc_sh.at[dst_v.at[j0]], add=True)
    pltpu.make_async_copy(g_sh.at[src_v.at[j0 + 1]], row1_v, sem1).wait()
    pltpu.sync_copy(row1_v, acc_sh.at[dst_v.at[j0 + 1]], add=True)

    plsc.subcore_barrier()
    pltpu.sync_copy(acc_sh.at[pl.ds(sid * RPT, RPT)],
                    out_hbm.at[cid, pl.ds(sid * RPT, RPT)])


def _dense_in_body(deg_ref, x_ref, w2_ref, g_ref, dis_ref):
    d = deg_ref[0] + deg_ref[1] + 1.0
    dis = lax.rsqrt(d)
    h = jnp.dot(x_ref[...], w2_ref[...], preferred_element_type=jnp.float32)
    g_ref[...] = h * dis
    dis_ref[...] = dis


_dense_in = pl.pallas_call(
    _dense_in_body,
    grid=(GRID,),
    in_specs=[
        pl.BlockSpec((NC, BLK, 1), lambda j: (0, j, 0)),
        pl.BlockSpec((BLK, F), lambda j: (j, 0)),
        pl.BlockSpec((F, C), lambda j: (0, 0)),
    ],
    out_specs=[
        pl.BlockSpec((BLK, C), lambda j: (j, 0)),
        pl.BlockSpec((BLK, 1), lambda j: (j, 0)),
    ],
    out_shape=[
        jax.ShapeDtypeStruct((N, C), jnp.float32),
        jax.ShapeDtypeStruct((N, 1), jnp.float32),
    ],
)


def _dense_out_body(s_ref, g_ref, dis_ref, b2_ref, wl_ref, bl_ref, o_ref):
    t = (s_ref[0] + s_ref[1] + g_ref[...]) * dis_ref[...] + b2_ref[...]
    t = jnp.maximum(t, 0.0)
    o_ref[...] = jnp.dot(t, wl_ref[...], preferred_element_type=jnp.float32) + bl_ref[...]


_dense_out = pl.pallas_call(
    _dense_out_body,
    grid=(GRID,),
    in_specs=[
        pl.BlockSpec((NC, BLK, C), lambda j: (0, j, 0)),
        pl.BlockSpec((BLK, C), lambda j: (j, 0)),
        pl.BlockSpec((BLK, 1), lambda j: (j, 0)),
        pl.BlockSpec((1, C), lambda j: (0, 0)),
        pl.BlockSpec((C, C), lambda j: (0, 0)),
        pl.BlockSpec((1, C), lambda j: (0, 0)),
    ],
    out_specs=pl.BlockSpec((BLK, C), lambda j: (j, 0)),
    out_shape=jax.ShapeDtypeStruct((N, C), jnp.float32),
)


def kernel(x, edge_index, W2, b2, Wl, bl):
    E = edge_index.shape[1]
    src = edge_index[0].astype(jnp.int32)
    dst = edge_index[1].astype(jnp.int32)
    pad = EPAD - E
    src_p = jnp.concatenate([src, jnp.zeros((pad,), jnp.int32)]).reshape(NW, NCH, CHUNK)
    dst_p = jnp.concatenate([dst, jnp.full((pad,), N, jnp.int32)]).reshape(NW, NCH, CHUNK)
    zo1 = jnp.stack([jnp.zeros((CHUNK, 1), jnp.float32),
                     jnp.ones((CHUNK, 1), jnp.float32)])
    zC = jnp.zeros((1, CHUNK, C), jnp.float32)
    deg2 = _degree_kernel(dst_p, zo1)
    g, dis = _dense_in(deg2, x, W2)
    s2 = _propagate_kernel(src_p, dst_p, g, zC)
    return _dense_out(s2, g, dis, b2.reshape(1, C), Wl, bl.reshape(1, C))


# speedup vs baseline: 67.4127x; 1.1088x over previous
import functools

import jax
import jax.numpy as jnp
from jax import lax
from jax.experimental import pallas as pl
from jax.experimental.pallas import tpu as pltpu
from jax.experimental.pallas import tpu_sc as plsc

N = 10000
F = 128
C = 16
NC = 2
NS = 16
NW = NC * NS
CHUNK = 125
NCH = 80
GPT = N // NS
BLK = 2000
GRID = N // BLK

_mesh = plsc.VectorSubcoreMesh(core_axis_name="c", subcore_axis_name="s")
_sc_params = pltpu.CompilerParams(use_tc_tiling_on_sc=False)


@functools.partial(
    pl.kernel,
    mesh=_mesh,
    out_type=jax.ShapeDtypeStruct((NC, N, 1), jnp.float32),
    compiler_params=_sc_params,
    scratch_types=[
        pltpu.VMEM((NCH, CHUNK), jnp.int32),
        pltpu.VMEM((CHUNK, 1), jnp.float32),
        pltpu.VMEM_SHARED((N, 1), jnp.float32),
        pltpu.SemaphoreType.DMA,
    ],
)
def _degree_kernel(e_hbm, zo_hbm, out_hbm, dst_v, buf_v, acc_sh, sem):
    cid = lax.axis_index("c")
    sid = lax.axis_index("s")
    wid = cid * NS + sid
    pltpu.sync_copy(zo_hbm.at[0], buf_v)
    for k in range(GPT // CHUNK):
        pltpu.sync_copy(buf_v, acc_sh.at[pl.ds(sid * GPT + k * CHUNK, CHUNK)])
    pltpu.sync_copy(zo_hbm.at[1], buf_v)
    pltpu.sync_copy(e_hbm.at[1, wid], dst_v)
    plsc.subcore_barrier()

    def body(j, carry):
        pltpu.sync_copy(buf_v, acc_sh.at[dst_v.at[j]], add=True)
        return carry

    lax.fori_loop(0, NCH, body, 0)
    plsc.subcore_barrier()
    @pl.when(sid < 10)
    def _copy_out():
        pltpu.sync_copy(acc_sh.at[pl.ds(sid * 1000, 1000)],
                        out_hbm.at[cid, pl.ds(sid * 1000, 1000)])


@functools.partial(
    pl.kernel,
    mesh=_mesh,
    out_type=jax.ShapeDtypeStruct((NC, N, C), jnp.float32),
    compiler_params=_sc_params,
    scratch_types=[
        pltpu.VMEM((NCH, CHUNK), jnp.int32),
        pltpu.VMEM((NCH, CHUNK), jnp.int32),
        pltpu.VMEM((8, CHUNK, C), jnp.float32),
        pltpu.VMEM((GPT, C), jnp.float32),
        pltpu.VMEM_SHARED((N, C), jnp.float32),
        pltpu.VMEM_SHARED((N, C), jnp.float32),
        pltpu.SemaphoreType.DMA,
        pltpu.SemaphoreType.DMA,
    ],
)
def _propagate_kernel(e_hbm, g_hbm, z_hbm, out_hbm,
                      src_v, dst_v, rows_v, stage_v, g_sh, acc_sh,
                      sem_g, sem_s):
    cid = lax.axis_index("c")
    sid = lax.axis_index("s")
    wid = cid * NS + sid
    pltpu.sync_copy(g_hbm.at[pl.ds(sid * GPT, GPT)], stage_v)
    pltpu.sync_copy(stage_v, g_sh.at[pl.ds(sid * GPT, GPT)])
    pltpu.sync_copy(z_hbm.at[0], rows_v.at[0])
    for k in range(GPT // CHUNK):
        pltpu.sync_copy(rows_v.at[0], acc_sh.at[pl.ds(sid * GPT + k * CHUNK, CHUNK)])
    pltpu.sync_copy(e_hbm.at[0, wid], src_v)
    pltpu.sync_copy(e_hbm.at[1, wid], dst_v)
    plsc.subcore_barrier()

    pltpu.async_copy(g_sh.at[src_v.at[0]], rows_v.at[0], sem_g)

    def body(i, carry):
        j0 = i * 2
        pltpu.make_async_copy(g_sh.at[src_v.at[j0]], rows_v.at[0], sem_g).wait()
        pltpu.async_copy(g_sh.at[src_v.at[j0 + 1]], rows_v.at[1], sem_s)
        pltpu.sync_copy(rows_v.at[0], acc_sh.at[dst_v.at[j0]], add=True)
        pltpu.make_async_copy(g_sh.at[src_v.at[j0 + 1]], rows_v.at[1], sem_s).wait()
        pltpu.async_copy(g_sh.at[src_v.at[j0 + 2]], rows_v.at[0], sem_g)
        pltpu.sync_copy(rows_v.at[1], acc_sh.at[dst_v.at[j0 + 1]], add=True)
        return carry

    lax.fori_loop(0, NCH // 2 - 1, body, 0)
    j0 = NCH - 2
    pltpu.make_async_copy(g_sh.at[src_v.at[j0]], rows_v.at[0], sem_g).wait()
    pltpu.async_copy(g_sh.at[src_v.at[j0 + 1]], rows_v.at[1], sem_s)
    pltpu.sync_copy(rows_v.at[0], acc_sh.at[dst_v.at[j0]], add=True)
    pltpu.make_async_copy(g_sh.at[src_v.at[j0 + 1]], rows_v.at[1], sem_s).wait()
    pltpu.sync_copy(rows_v.at[1], acc_sh.at[dst_v.at[j0 + 1]], add=True)

    plsc.subcore_barrier()
    pltpu.sync_copy(acc_sh.at[pl.ds(sid * GPT, GPT)],
                    out_hbm.at[cid, pl.ds(sid * GPT, GPT)])


def _dense_in_body(deg_ref, x_ref, w2_ref, g_ref, dis_ref):
    d = deg_ref[0] + deg_ref[1] + 1.0
    dis = lax.rsqrt(d)
    h = jnp.dot(x_ref[...], w2_ref[...], preferred_element_type=jnp.float32)
    g_ref[...] = h * dis
    dis_ref[...] = dis


_dense_in = pl.pallas_call(
    _dense_in_body,
    grid=(GRID,),
    in_specs=[
        pl.BlockSpec((NC, BLK, 1), lambda j: (0, j, 0)),
        pl.BlockSpec((BLK, F), lambda j: (j, 0)),
        pl.BlockSpec((F, C), lambda j: (0, 0)),
    ],
    out_specs=[
        pl.BlockSpec((BLK, C), lambda j: (j, 0)),
        pl.BlockSpec((BLK, 1), lambda j: (j, 0)),
    ],
    out_shape=[
        jax.ShapeDtypeStruct((N, C), jnp.float32),
        jax.ShapeDtypeStruct((N, 1), jnp.float32),
    ],
)


def _dense_out_body(s_ref, g_ref, dis_ref, b2_ref, wl_ref, bl_ref, o_ref):
    t = (s_ref[0] + s_ref[1] + g_ref[...]) * dis_ref[...] + b2_ref[...]
    t = jnp.maximum(t, 0.0)
    o_ref[...] = jnp.dot(t, wl_ref[...], preferred_element_type=jnp.float32) + bl_ref[...]


_dense_out = pl.pallas_call(
    _dense_out_body,
    grid=(GRID,),
    in_specs=[
        pl.BlockSpec((NC, BLK, C), lambda j: (0, j, 0)),
        pl.BlockSpec((BLK, C), lambda j: (j, 0)),
        pl.BlockSpec((BLK, 1), lambda j: (j, 0)),
        pl.BlockSpec((1, C), lambda j: (0, 0)),
        pl.BlockSpec((C, C), lambda j: (0, 0)),
        pl.BlockSpec((1, C), lambda j: (0, 0)),
    ],
    out_specs=pl.BlockSpec((BLK, C), lambda j: (j, 0)),
    out_shape=jax.ShapeDtypeStruct((N, C), jnp.float32),
)


def kernel(x, edge_index, W2, b2, Wl, bl):
    e_r = edge_index.astype(jnp.int32).reshape(2, NW, NCH, CHUNK)
    zo1 = jnp.stack([jnp.zeros((CHUNK, 1), jnp.float32),
                     jnp.ones((CHUNK, 1), jnp.float32)])
    zC = jnp.zeros((1, CHUNK, C), jnp.float32)
    deg2 = _degree_kernel(e_r, zo1)
    g, dis = _dense_in(deg2, x, W2)
    s2 = _propagate_kernel(e_r, g, zC)
    return _dense_out(s2, g, dis, b2.reshape(1, C), Wl, bl.reshape(1, C))


# speedup vs baseline: 70.0511x; 1.0391x over previous
import functools

import jax
import jax.numpy as jnp
from jax import lax
from jax.experimental import pallas as pl
from jax.experimental.pallas import tpu as pltpu
from jax.experimental.pallas import tpu_sc as plsc

N = 10000
F = 128
C = 16
NC = 2
NS = 16
NW = NC * NS
CHUNK = 125
NCH = 80
GPT = N // NS
BLK = 2000
GRID = N // BLK

_mesh = plsc.VectorSubcoreMesh(core_axis_name="c", subcore_axis_name="s")
_sc_params = pltpu.CompilerParams(use_tc_tiling_on_sc=False)


@functools.partial(
    pl.kernel,
    mesh=_mesh,
    out_type=jax.ShapeDtypeStruct((NC, N, 1), jnp.float32),
    compiler_params=_sc_params,
    scratch_types=[
        pltpu.VMEM((NCH, CHUNK), jnp.int32),
        pltpu.VMEM((CHUNK, 1), jnp.float32),
        pltpu.VMEM_SHARED((N, 1), jnp.float32),
    ] + [pltpu.SemaphoreType.DMA] * 8,
)
def _degree_kernel(e_hbm, zo_hbm, out_hbm, dst_v, buf_v, acc_sh, *sems):
    cid = lax.axis_index("c")
    sid = lax.axis_index("s")
    wid = cid * NS + sid
    pltpu.sync_copy(zo_hbm.at[0], buf_v)
    for k in range(GPT // CHUNK):
        pltpu.sync_copy(buf_v, acc_sh.at[pl.ds(sid * GPT + k * CHUNK, CHUNK)])
    pltpu.sync_copy(zo_hbm.at[1], buf_v)
    pltpu.sync_copy(e_hbm.at[1, wid], dst_v)
    plsc.subcore_barrier()

    def body(j, carry):
        pltpu.sync_copy(buf_v, acc_sh.at[dst_v.at[j]], add=True)
        return carry

    lax.fori_loop(0, NCH, body, 0)
    plsc.subcore_barrier()
    @pl.when(sid < 10)
    def _copy_out():
        pltpu.sync_copy(acc_sh.at[pl.ds(sid * 1000, 1000)],
                        out_hbm.at[cid, pl.ds(sid * 1000, 1000)])


@functools.partial(
    pl.kernel,
    mesh=_mesh,
    out_type=jax.ShapeDtypeStruct((NC, N, C), jnp.float32),
    compiler_params=_sc_params,
    scratch_types=[
        pltpu.VMEM((NCH, CHUNK), jnp.int32),
        pltpu.VMEM((NCH, CHUNK), jnp.int32),
        pltpu.VMEM((8, CHUNK, C), jnp.float32),
        pltpu.VMEM((GPT, C), jnp.float32),
        pltpu.VMEM_SHARED((N, C), jnp.float32),
        pltpu.VMEM_SHARED((N, C), jnp.float32),
    ] + [pltpu.SemaphoreType.DMA] * 16,
)
def _propagate_kernel(e_hbm, g_hbm, z_hbm, out_hbm,
                      src_v, dst_v, rows_v, stage_v, g_sh, acc_sh,
                      *sems):
    cid = lax.axis_index("c")
    sid = lax.axis_index("s")
    wid = cid * NS + sid
    pltpu.sync_copy(g_hbm.at[pl.ds(sid * GPT, GPT)], stage_v)
    pltpu.sync_copy(stage_v, g_sh.at[pl.ds(sid * GPT, GPT)])
    pltpu.sync_copy(z_hbm.at[0], rows_v.at[0])
    for k in range(GPT // CHUNK):
        pltpu.sync_copy(rows_v.at[0], acc_sh.at[pl.ds(sid * GPT + k * CHUNK, CHUNK)])
    pltpu.sync_copy(e_hbm.at[0, wid], src_v)
    pltpu.sync_copy(e_hbm.at[1, wid], dst_v)
    plsc.subcore_barrier()

    sem_g = sems[:8]
    sem_s = sems[8:]
    for b in range(4):
        pltpu.async_copy(g_sh.at[src_v.at[b]], rows_v.at[b], sem_g[b])

    def body(i, carry):
        for b in range(8):
            j = i * 8 + b
            bp = (b + 4) % 8
            pltpu.make_async_copy(g_sh.at[src_v.at[j]], rows_v.at[b], sem_g[b]).wait()
            pltpu.async_copy(rows_v.at[b], acc_sh.at[dst_v.at[j]], sem_s[b], add=True)
            @pl.when(j >= 4)
            def _drain():
                pltpu.make_async_copy(rows_v.at[bp], acc_sh.at[dst_v.at[j - 4]], sem_s[bp]).wait()
            @pl.when(j < NCH - 4)
            def _prefetch():
                pltpu.async_copy(g_sh.at[src_v.at[j + 4]], rows_v.at[bp], sem_g[bp])
        return carry

    lax.fori_loop(0, NCH // 8, body, 0)
    for jj in range(NCH - 4, NCH):
        b = jj % 8
        pltpu.make_async_copy(rows_v.at[b], acc_sh.at[dst_v.at[jj]], sem_s[b]).wait()

    plsc.subcore_barrier()
    pltpu.sync_copy(acc_sh.at[pl.ds(sid * GPT, GPT)],
                    out_hbm.at[cid, pl.ds(sid * GPT, GPT)])


def _dense_in_body(deg_ref, x_ref, w2_ref, g_ref, dis_ref):
    d = deg_ref[0] + deg_ref[1] + 1.0
    dis = lax.rsqrt(d)
    h = jnp.dot(x_ref[...], w2_ref[...], preferred_element_type=jnp.float32)
    g_ref[...] = h * dis
    dis_ref[...] = dis


_dense_in = pl.pallas_call(
    _dense_in_body,
    grid=(GRID,),
    in_specs=[
        pl.BlockSpec((NC, BLK, 1), lambda j: (0, j, 0)),
        pl.BlockSpec((BLK, F), lambda j: (j, 0)),
        pl.BlockSpec((F, C), lambda j: (0, 0)),
    ],
    out_specs=[
        pl.BlockSpec((BLK, C), lambda j: (j, 0)),
        pl.BlockSpec((BLK, 1), lambda j: (j, 0)),
    ],
    out_shape=[
        jax.ShapeDtypeStruct((N, C), jnp.float32),
        jax.ShapeDtypeStruct((N, 1), jnp.float32),
    ],
)


def _dense_out_body(s_ref, g_ref, dis_ref, b2_ref, wl_ref, bl_ref, o_ref):
    t = (s_ref[0] + s_ref[1] + g_ref[...]) * dis_ref[...] + b2_ref[...]
    t = jnp.maximum(t, 0.0)
    o_ref[...] = jnp.dot(t, wl_ref[...], preferred_element_type=jnp.float32) + bl_ref[...]


_dense_out = pl.pallas_call(
    _dense_out_body,
    grid=(GRID,),
    in_specs=[
        pl.BlockSpec((NC, BLK, C), lambda j: (0, j, 0)),
        pl.BlockSpec((BLK, C), lambda j: (j, 0)),
        pl.BlockSpec((BLK, 1), lambda j: (j, 0)),
        pl.BlockSpec((1, C), lambda j: (0, 0)),
        pl.BlockSpec((C, C), lambda j: (0, 0)),
        pl.BlockSpec((1, C), lambda j: (0, 0)),
    ],
    out_specs=pl.BlockSpec((BLK, C), lambda j: (j, 0)),
    out_shape=jax.ShapeDtypeStruct((N, C), jnp.float32),
)


def kernel(x, edge_index, W2, b2, Wl, bl):
    e_r = edge_index.astype(jnp.int32).reshape(2, NW, NCH, CHUNK)
    zo1 = jnp.stack([jnp.zeros((CHUNK, 1), jnp.float32),
                     jnp.ones((CHUNK, 1), jnp.float32)])
    zC = jnp.zeros((1, CHUNK, C), jnp.float32)
    deg2 = _degree_kernel(e_r, zo1)
    g, dis = _dense_in(deg2, x, W2)
    s2 = _propagate_kernel(e_r, g, zC)
    return _dense_out(s2, g, dis, b2.reshape(1, C), Wl, bl.reshape(1, C))


# speedup vs baseline: 73.5242x; 1.0496x over previous
import functools

import jax
import jax.numpy as jnp
from jax import lax
from jax.experimental import pallas as pl
from jax.experimental.pallas import tpu as pltpu
from jax.experimental.pallas import tpu_sc as plsc

N = 10000
F = 128
C = 16
NC = 2
NS = 16
NW = NC * NS
CHUNK = 125
NCH = 80
GPT = N // NS
BLK = 2000
GRID = N // BLK

_mesh = plsc.VectorSubcoreMesh(core_axis_name="c", subcore_axis_name="s")
_sc_params = pltpu.CompilerParams(use_tc_tiling_on_sc=False)


@functools.partial(
    pl.kernel,
    mesh=_mesh,
    out_type=jax.ShapeDtypeStruct((NC, N, 1), jnp.float32),
    compiler_params=_sc_params,
    scratch_types=[
        pltpu.VMEM((NCH, CHUNK), jnp.int32),
        pltpu.VMEM((CHUNK, 1), jnp.float32),
        pltpu.VMEM_SHARED((N, 1), jnp.float32),
    ] + [pltpu.SemaphoreType.DMA] * 8,
)
def _degree_kernel(e_hbm, zo_hbm, out_hbm, dst_v, buf_v, acc_sh, *sems):
    cid = lax.axis_index("c")
    sid = lax.axis_index("s")
    wid = cid * NS + sid
    pltpu.sync_copy(zo_hbm.at[0], buf_v)
    for k in range(GPT // CHUNK):
        pltpu.sync_copy(buf_v, acc_sh.at[pl.ds(sid * GPT + k * CHUNK, CHUNK)])
    pltpu.sync_copy(zo_hbm.at[1], buf_v)
    pltpu.sync_copy(e_hbm.at[1, wid], dst_v)
    plsc.subcore_barrier()

    def body(i, carry):
        for b in range(8):
            j = i * 8 + b
            @pl.when(j >= 8)
            def _drain():
                pltpu.make_async_copy(buf_v, acc_sh.at[dst_v.at[j - 8]], sems[b]).wait()
            pltpu.async_copy(buf_v, acc_sh.at[dst_v.at[j]], sems[b], add=True)
        return carry

    lax.fori_loop(0, NCH // 8, body, 0)
    for jj in range(NCH - 8, NCH):
        pltpu.make_async_copy(buf_v, acc_sh.at[dst_v.at[jj]], sems[jj % 8]).wait()
    plsc.subcore_barrier()
    @pl.when(sid < 10)
    def _copy_out():
        pltpu.sync_copy(acc_sh.at[pl.ds(sid * 1000, 1000)],
                        out_hbm.at[cid, pl.ds(sid * 1000, 1000)])


@functools.partial(
    pl.kernel,
    mesh=_mesh,
    out_type=jax.ShapeDtypeStruct((NC, N, C), jnp.float32),
    compiler_params=_sc_params,
    scratch_types=[
        pltpu.VMEM((NCH, CHUNK), jnp.int32),
        pltpu.VMEM((NCH, CHUNK), jnp.int32),
        pltpu.VMEM((8, CHUNK, C), jnp.float32),
        pltpu.VMEM((GPT, C), jnp.float32),
        pltpu.VMEM_SHARED((N, C), jnp.float32),
        pltpu.VMEM_SHARED((N, C), jnp.float32),
    ] + [pltpu.SemaphoreType.DMA] * 16,
)
def _propagate_kernel(e_hbm, g_hbm, z_hbm, out_hbm,
                      src_v, dst_v, rows_v, stage_v, g_sh, acc_sh,
                      *sems):
    cid = lax.axis_index("c")
    sid = lax.axis_index("s")
    wid = cid * NS + sid
    pltpu.sync_copy(g_hbm.at[pl.ds(sid * GPT, GPT)], stage_v)
    pltpu.sync_copy(stage_v, g_sh.at[pl.ds(sid * GPT, GPT)])
    pltpu.sync_copy(z_hbm.at[0], rows_v.at[0])
    for k in range(GPT // CHUNK):
        pltpu.sync_copy(rows_v.at[0], acc_sh.at[pl.ds(sid * GPT + k * CHUNK, CHUNK)])
    pltpu.sync_copy(e_hbm.at[0, wid], src_v)
    pltpu.sync_copy(e_hbm.at[1, wid], dst_v)
    plsc.subcore_barrier()

    sem_g = sems[:8]
    sem_s = sems[8:]
    for b in range(4):
        pltpu.async_copy(g_sh.at[src_v.at[b]], rows_v.at[b], sem_g[b])

    def body(i, carry):
        for b in range(8):
            j = i * 8 + b
            bp = (b + 4) % 8
            pltpu.make_async_copy(g_sh.at[src_v.at[j]], rows_v.at[b], sem_g[b]).wait()
            pltpu.async_copy(rows_v.at[b], acc_sh.at[dst_v.at[j]], sem_s[b], add=True)
            @pl.when(j >= 4)
            def _drain():
                pltpu.make_async_copy(rows_v.at[bp], acc_sh.at[dst_v.at[j - 4]], sem_s[bp]).wait()
            @pl.when(j < NCH - 4)
            def _prefetch():
                pltpu.async_copy(g_sh.at[src_v.at[j + 4]], rows_v.at[bp], sem_g[bp])
        return carry

    lax.fori_loop(0, NCH // 8, body, 0)
    for jj in range(NCH - 4, NCH):
        b = jj % 8
        pltpu.make_async_copy(rows_v.at[b], acc_sh.at[dst_v.at[jj]], sem_s[b]).wait()

    plsc.subcore_barrier()
    pltpu.sync_copy(acc_sh.at[pl.ds(sid * GPT, GPT)],
                    out_hbm.at[cid, pl.ds(sid * GPT, GPT)])


def _dense_in_body(deg_ref, x_ref, w2_ref, g_ref, dis_ref):
    d = deg_ref[0] + deg_ref[1] + 1.0
    dis = lax.rsqrt(d)
    h = jnp.dot(x_ref[...], w2_ref[...], preferred_element_type=jnp.float32)
    g_ref[...] = h * dis
    dis_ref[...] = dis


_dense_in = pl.pallas_call(
    _dense_in_body,
    grid=(GRID,),
    in_specs=[
        pl.BlockSpec((NC, BLK, 1), lambda j: (0, j, 0)),
        pl.BlockSpec((BLK, F), lambda j: (j, 0)),
        pl.BlockSpec((F, C), lambda j: (0, 0)),
    ],
    out_specs=[
        pl.BlockSpec((BLK, C), lambda j: (j, 0)),
        pl.BlockSpec((BLK, 1), lambda j: (j, 0)),
    ],
    out_shape=[
        jax.ShapeDtypeStruct((N, C), jnp.float32),
        jax.ShapeDtypeStruct((N, 1), jnp.float32),
    ],
)


def _dense_out_body(s_ref, g_ref, dis_ref, b2_ref, wl_ref, bl_ref, o_ref):
    t = (s_ref[0] + s_ref[1] + g_ref[...]) * dis_ref[...] + b2_ref[...]
    t = jnp.maximum(t, 0.0)
    o_ref[...] = jnp.dot(t, wl_ref[...], preferred_element_type=jnp.float32) + bl_ref[...]


_dense_out = pl.pallas_call(
    _dense_out_body,
    grid=(GRID,),
    in_specs=[
        pl.BlockSpec((NC, BLK, C), lambda j: (0, j, 0)),
        pl.BlockSpec((BLK, C), lambda j: (j, 0)),
        pl.BlockSpec((BLK, 1), lambda j: (j, 0)),
        pl.BlockSpec((1, C), lambda j: (0, 0)),
        pl.BlockSpec((C, C), lambda j: (0, 0)),
        pl.BlockSpec((1, C), lambda j: (0, 0)),
    ],
    out_specs=pl.BlockSpec((BLK, C), lambda j: (j, 0)),
    out_shape=jax.ShapeDtypeStruct((N, C), jnp.float32),
)


def kernel(x, edge_index, W2, b2, Wl, bl):
    e_r = edge_index.astype(jnp.int32).reshape(2, NW, NCH, CHUNK)
    zo1 = jnp.stack([jnp.zeros((CHUNK, 1), jnp.float32),
                     jnp.ones((CHUNK, 1), jnp.float32)])
    zC = jnp.zeros((1, CHUNK, C), jnp.float32)
    deg2 = _degree_kernel(e_r, zo1)
    g, dis = _dense_in(deg2, x, W2)
    s2 = _propagate_kernel(e_r, g, zC)
    return _dense_out(s2, g, dis, b2.reshape(1, C), Wl, bl.reshape(1, C))


# speedup vs baseline: 90.0170x; 1.2243x over previous
import functools

import jax
import jax.numpy as jnp
from jax import lax
from jax.experimental import pallas as pl
from jax.experimental.pallas import tpu as pltpu
from jax.experimental.pallas import tpu_sc as plsc

N = 10000
F = 128
C = 16
NC = 2
NS = 16
NW = NC * NS
CHUNK = 125
NCH = 80
NPAD = 10240
RPT = NPAD // NS
NLAST = N - (NS - 1) * RPT
BLK = 2000
GRID = N // BLK
PR = NPAD * C // 128
PBLK = PR // 5

_mesh = plsc.VectorSubcoreMesh(core_axis_name="c", subcore_axis_name="s")
_sc_params = pltpu.CompilerParams(use_tc_tiling_on_sc=False,
                                  needs_layout_passes=False)


@functools.partial(
    pl.kernel,
    mesh=_mesh,
    out_type=jax.ShapeDtypeStruct((NC, NPAD), jnp.float32),
    compiler_params=_sc_params,
    scratch_types=[
        pltpu.VMEM((NCH, CHUNK), jnp.int32),
        pltpu.VMEM((160,), jnp.float32),
        pltpu.VMEM((CHUNK,), jnp.float32),
        pltpu.VMEM_SHARED((NPAD,), jnp.float32),
    ] + [pltpu.SemaphoreType.DMA] * 8,
)
def _degree_kernel(e_hbm, z_hbm, o_hbm, out_hbm, dst_v, zbuf_v, ones_v, acc_sh, *sems):
    cid = lax.axis_index("c")
    sid = lax.axis_index("s")
    wid = cid * NS + sid
    pltpu.sync_copy(z_hbm, zbuf_v)
    for k in range(RPT // 160):
        pltpu.sync_copy(zbuf_v, acc_sh.at[pl.ds(sid * RPT + k * 160, 160)])
    pltpu.sync_copy(o_hbm, ones_v)
    pltpu.sync_copy(e_hbm.at[1, wid], dst_v)
    plsc.subcore_barrier()

    def body(i, carry):
        for b in range(8):
            j = i * 8 + b
            @pl.when(j >= 8)
            def _drain():
                pltpu.make_async_copy(ones_v, acc_sh.at[dst_v.at[j - 8]], sems[b]).wait()
            pltpu.async_copy(ones_v, acc_sh.at[dst_v.at[j]], sems[b], add=True)
        return carry

    lax.fori_loop(0, NCH // 8, body, 0)
    for jj in range(NCH - 8, NCH):
        pltpu.make_async_copy(ones_v, acc_sh.at[dst_v.at[jj]], sems[jj % 8]).wait()
    plsc.subcore_barrier()
    pltpu.sync_copy(acc_sh.at[pl.ds(sid * RPT, RPT)],
                    out_hbm.at[cid, pl.ds(sid * RPT, RPT)])


def _rsqrt_newton(d):
    i = plsc.bitcast(d, jnp.int32)
    i = jnp.int32(0x5F3759DF) - lax.shift_right_arithmetic(i, 1)
    y = plsc.bitcast(i, jnp.float32)
    for _ in range(3):
        y = y * (1.5 - 0.5 * d * y * y)
    return y


@functools.partial(
    pl.kernel,
    mesh=_mesh,
    out_type=jax.ShapeDtypeStruct((NC, NPAD, C), jnp.float32),
    compiler_params=_sc_params,
    scratch_types=[
        pltpu.VMEM((NCH, CHUNK), jnp.int32),
        pltpu.VMEM((NCH, CHUNK), jnp.int32),
        pltpu.VMEM((8, CHUNK, C), jnp.float32),
        pltpu.VMEM((RPT, C), jnp.float32),
        pltpu.VMEM((RPT, C), jnp.float32),
        pltpu.VMEM((RPT,), jnp.float32),
        pltpu.VMEM((RPT,), jnp.float32),
        pltpu.VMEM((RPT,), jnp.float32),
        pltpu.VMEM_SHARED((NPAD, C), jnp.float32),
        pltpu.VMEM_SHARED((NPAD, C), jnp.float32),
    ] + [pltpu.SemaphoreType.DMA] * 16,
)
def _propagate_kernel(e_hbm, h_hbm, deg_hbm, out_hbm,
                      src_v, dst_v, rows_v, stage_v, s_v,
                      deg0_v, deg1_v, dis_v, g_sh, acc_sh, *sems):
    cid = lax.axis_index("c")
    sid = lax.axis_index("s")
    wid = cid * NS + sid
    sem_g = sems[:8]
    sem_s = sems[8:]

    @pl.when(sid < NS - 1)
    def _full():
        pltpu.sync_copy(h_hbm.at[pl.ds(sid * RPT, RPT)], stage_v)
    @pl.when(sid == NS - 1)
    def _tail():
        pltpu.sync_copy(h_hbm.at[pl.ds((NS - 1) * RPT, NLAST)],
                        stage_v.at[pl.ds(0, NLAST)])
    pltpu.sync_copy(deg_hbm.at[0, pl.ds(sid * RPT, RPT)], deg0_v)
    pltpu.sync_copy(deg_hbm.at[1, pl.ds(sid * RPT, RPT)], deg1_v)

    def zbody(i, carry):
        s_v[i, :] = jnp.zeros((C,), jnp.float32)
        return carry
    lax.fori_loop(0, RPT, zbody, 0)
    pltpu.sync_copy(s_v, acc_sh.at[pl.ds(sid * RPT, RPT)])

    def dbody(k, carry):
        d = deg0_v[pl.ds(k * C, C)] + deg1_v[pl.ds(k * C, C)] + 1.0
        dis_v[pl.ds(k * C, C)] = _rsqrt_newton(d)
        return carry
    lax.fori_loop(0, RPT // C, dbody, 0)

    def gbody(i, carry):
        sp = plsc.load_gather(dis_v, [jnp.full((C,), i, jnp.int32)])
        stage_v[i, :] = stage_v[i, :] * sp
        return carry
    lax.fori_loop(0, RPT, gbody, 0)
    pltpu.sync_copy(stage_v, g_sh.at[pl.ds(sid * RPT, RPT)])

    pltpu.sync_copy(e_hbm.at[0, wid], src_v)
    pltpu.sync_copy(e_hbm.at[1, wid], dst_v)
    plsc.subcore_barrier()

    for b in range(4):
        pltpu.async_copy(g_sh.at[src_v.at[b]], rows_v.at[b], sem_g[b])

    def body(i, carry):
        for b in range(8):
            j = i * 8 + b
            bp = (b + 4) % 8
            pltpu.make_async_copy(g_sh.at[src_v.at[j]], rows_v.at[b], sem_g[b]).wait()
            pltpu.async_copy(rows_v.at[b], acc_sh.at[dst_v.at[j]], sem_s[b], add=True)
            @pl.when(j >= 4)
            def _drain():
                pltpu.make_async_copy(rows_v.at[bp], acc_sh.at[dst_v.at[j - 4]], sem_s[bp]).wait()
            @pl.when(j < NCH - 4)
            def _prefetch():
                pltpu.async_copy(g_sh.at[src_v.at[j + 4]], rows_v.at[bp], sem_g[bp])
        return carry

    lax.fori_loop(0, NCH // 8, body, 0)
    for jj in range(NCH - 4, NCH):
        b = jj % 8
        pltpu.make_async_copy(rows_v.at[b], acc_sh.at[dst_v.at[jj]], sem_s[b]).wait()
    plsc.subcore_barrier()

    pltpu.sync_copy(acc_sh.at[pl.ds(sid * RPT, RPT)], s_v)
    w = jnp.where(cid == 0, 1.0, 0.0).astype(jnp.float32)
    def obody(i, carry):
        sp = plsc.load_gather(dis_v, [jnp.full((C,), i, jnp.int32)])
        s_v[i, :] = (s_v[i, :] + w * stage_v[i, :]) * sp
        return carry
    lax.fori_loop(0, RPT, obody, 0)
    pltpu.sync_copy(s_v, out_hbm.at[cid, pl.ds(sid * RPT, RPT)])


def _matmul_h_body(x_ref, w2_ref, h_ref):
    h_ref[...] = jnp.dot(x_ref[...], w2_ref[...], preferred_element_type=jnp.float32)


_matmul_h = pl.pallas_call(
    _matmul_h_body,
    grid=(GRID,),
    in_specs=[
        pl.BlockSpec((BLK, F), lambda j: (j, 0)),
        pl.BlockSpec((F, C), lambda j: (0, 0)),
    ],
    out_specs=pl.BlockSpec((BLK, C), lambda j: (j, 0)),
    out_shape=jax.ShapeDtypeStruct((N, C), jnp.float32),
)


def _dense_out_body(z_ref, b2_ref, wl_ref, bl_ref, o_ref):
    t = jnp.maximum(z_ref[0] + z_ref[1] + b2_ref[...], 0.0)
    o_ref[...] = jnp.dot(t, wl_ref[...], preferred_element_type=jnp.float32) + bl_ref[...]


_dense_out = pl.pallas_call(
    _dense_out_body,
    grid=(5,),
    in_specs=[
        pl.BlockSpec((NC, PBLK, 128), lambda j: (0, j, 0)),
        pl.BlockSpec((1, 128), lambda j: (0, 0)),
        pl.BlockSpec((128, 128), lambda j: (0, 0)),
        pl.BlockSpec((1, 128), lambda j: (0, 0)),
    ],
    out_specs=pl.BlockSpec((PBLK, 128), lambda j: (j, 0)),
    out_shape=jax.ShapeDtypeStruct((PR, 128), jnp.float32),
)


def kernel(x, edge_index, W2, b2, Wl, bl):
    e_r = edge_index.astype(jnp.int32).reshape(2, NW, NCH, CHUNK)
    deg2 = _degree_kernel(e_r, jnp.zeros((160,), jnp.float32),
                          jnp.ones((CHUNK,), jnp.float32))
    h = _matmul_h(x, W2)
    z2 = _propagate_kernel(e_r, h, deg2)
    z2p = z2.reshape(NC, PR, 128)
    wlk = jnp.kron(jnp.eye(8, dtype=jnp.float32), Wl)
    b2p = jnp.tile(b2, 8).reshape(1, 128)
    blp = jnp.tile(bl, 8).reshape(1, 128)
    outw = _dense_out(z2p, b2p, wlk, blp)
    return outw.reshape(NPAD, C)[:N]


# speedup vs baseline: 91.5589x; 1.0171x over previous
import functools

import jax
import jax.numpy as jnp
from jax import lax
from jax.experimental import pallas as pl
from jax.experimental.pallas import tpu as pltpu
from jax.experimental.pallas import tpu_sc as plsc

N = 10000
F = 128
C = 16
NC = 2
NS = 16
NW = NC * NS
CHUNK = 125
NCH = 80
NPAD = 10240
RPT = NPAD // NS
NLAST = N - (NS - 1) * RPT
BLK = 2000
GRID = N // BLK
PR = NPAD * C // 128
PBLK = PR // 5

_mesh = plsc.VectorSubcoreMesh(core_axis_name="c", subcore_axis_name="s")
_sc_params = pltpu.CompilerParams(use_tc_tiling_on_sc=False,
                                  needs_layout_passes=False)


@functools.partial(
    pl.kernel,
    mesh=_mesh,
    out_type=jax.ShapeDtypeStruct((NC, NPAD), jnp.float32),
    compiler_params=_sc_params,
    scratch_types=[
        pltpu.VMEM((NCH, CHUNK), jnp.int32),
        pltpu.VMEM((160,), jnp.float32),
        pltpu.VMEM((CHUNK,), jnp.float32),
        pltpu.VMEM_SHARED((NPAD,), jnp.float32),
    ] + [pltpu.SemaphoreType.DMA] * 8,
)
def _degree_kernel(e_hbm, z_hbm, o_hbm, out_hbm, dst_v, zbuf_v, ones_v, acc_sh, *sems):
    cid = lax.axis_index("c")
    sid = lax.axis_index("s")
    wid = cid * NS + sid
    pltpu.sync_copy(z_hbm, zbuf_v)
    for k in range(RPT // 160):
        pltpu.sync_copy(zbuf_v, acc_sh.at[pl.ds(sid * RPT + k * 160, 160)])
    pltpu.sync_copy(o_hbm, ones_v)
    pltpu.sync_copy(e_hbm.at[1, wid], dst_v)
    plsc.subcore_barrier()

    def body(i, carry):
        for b in range(8):
            j = i * 8 + b
            @pl.when(j >= 8)
            def _drain():
                pltpu.make_async_copy(ones_v, acc_sh.at[dst_v.at[j - 8]], sems[b]).wait()
            pltpu.async_copy(ones_v, acc_sh.at[dst_v.at[j]], sems[b], add=True)
        return carry

    lax.fori_loop(0, NCH // 8, body, 0)
    for jj in range(NCH - 8, NCH):
        pltpu.make_async_copy(ones_v, acc_sh.at[dst_v.at[jj]], sems[jj % 8]).wait()
    plsc.subcore_barrier()
    pltpu.sync_copy(acc_sh.at[pl.ds(sid * RPT, RPT)],
                    out_hbm.at[cid, pl.ds(sid * RPT, RPT)])


def _rsqrt_newton(d):
    i = plsc.bitcast(d, jnp.int32)
    i = jnp.int32(0x5F3759DF) - lax.shift_right_arithmetic(i, 1)
    y = plsc.bitcast(i, jnp.float32)
    for _ in range(3):
        y = y * (1.5 - 0.5 * d * y * y)
    return y


@functools.partial(
    pl.kernel,
    mesh=_mesh,
    out_type=jax.ShapeDtypeStruct((NC, N, C), jnp.float32),
    compiler_params=_sc_params,
    scratch_types=[
        pltpu.VMEM((NCH, CHUNK), jnp.int32),
        pltpu.VMEM((NCH, CHUNK), jnp.int32),
        pltpu.VMEM((8, CHUNK, C), jnp.float32),
        pltpu.VMEM((RPT, C), jnp.float32),
        pltpu.VMEM((RPT, C), jnp.float32),
        pltpu.VMEM((RPT,), jnp.float32),
        pltpu.VMEM((RPT,), jnp.float32),
        pltpu.VMEM((RPT,), jnp.float32),
        pltpu.VMEM_SHARED((N, C), jnp.float32),
        pltpu.VMEM_SHARED((N, C), jnp.float32),
    ] + [pltpu.SemaphoreType.DMA] * 16,
)
def _propagate_kernel(e_hbm, h_hbm, deg_hbm, z_hbm, out_hbm,
                      src_v, dst_v, rows_v, stage_v, s_v,
                      deg0_v, deg1_v, dis_v, g_sh, acc_sh, *sems):
    cid = lax.axis_index("c")
    sid = lax.axis_index("s")
    wid = cid * NS + sid
    sem_g = sems[:8]
    sem_s = sems[8:]

    @pl.when(sid < NS - 1)
    def _full():
        pltpu.sync_copy(h_hbm.at[pl.ds(sid * RPT, RPT)], stage_v)
    @pl.when(sid == NS - 1)
    def _tail():
        pltpu.sync_copy(h_hbm.at[pl.ds((NS - 1) * RPT, NLAST)],
                        stage_v.at[pl.ds(0, NLAST)])
    pltpu.sync_copy(deg_hbm.at[0, pl.ds(sid * RPT, RPT)], deg0_v)
    pltpu.sync_copy(deg_hbm.at[1, pl.ds(sid * RPT, RPT)], deg1_v)

    @pl.when(sid < NS - 1)
    def _zfull():
        pltpu.sync_copy(z_hbm, acc_sh.at[pl.ds(sid * RPT, RPT)])
    @pl.when(sid == NS - 1)
    def _ztail():
        pltpu.sync_copy(z_hbm.at[pl.ds(0, NLAST)],
                        acc_sh.at[pl.ds((NS - 1) * RPT, NLAST)])

    def dbody(k, carry):
        d = deg0_v[pl.ds(k * C, C)] + deg1_v[pl.ds(k * C, C)] + 1.0
        dis_v[pl.ds(k * C, C)] = _rsqrt_newton(d)
        return carry
    lax.fori_loop(0, RPT // C, dbody, 0)

    def gbody(i, carry):
        sp = plsc.load_gather(dis_v, [jnp.full((C,), i, jnp.int32)])
        stage_v[i, :] = stage_v[i, :] * sp
        return carry
    lax.fori_loop(0, RPT, gbody, 0)
    @pl.when(sid < NS - 1)
    def _gfull():
        pltpu.sync_copy(stage_v, g_sh.at[pl.ds(sid * RPT, RPT)])
    @pl.when(sid == NS - 1)
    def _gtail():
        pltpu.sync_copy(stage_v.at[pl.ds(0, NLAST)],
                        g_sh.at[pl.ds((NS - 1) * RPT, NLAST)])

    pltpu.sync_copy(e_hbm.at[0, wid], src_v)
    pltpu.sync_copy(e_hbm.at[1, wid], dst_v)
    plsc.subcore_barrier()

    for b in range(4):
        pltpu.async_copy(g_sh.at[src_v.at[b]], rows_v.at[b], sem_g[b])

    def body(i, carry):
        for b in range(8):
            j = i * 8 + b
            bp = (b + 4) % 8
            pltpu.make_async_copy(g_sh.at[src_v.at[j]], rows_v.at[b], sem_g[b]).wait()
            pltpu.async_copy(rows_v.at[b], acc_sh.at[dst_v.at[j]], sem_s[b], add=True)
            @pl.when(j >= 4)
            def _drain():
                pltpu.make_async_copy(rows_v.at[bp], acc_sh.at[dst_v.at[j - 4]], sem_s[bp]).wait()
            @pl.when(j < NCH - 4)
            def _prefetch():
                pltpu.async_copy(g_sh.at[src_v.at[j + 4]], rows_v.at[bp], sem_g[bp])
        return carry

    lax.fori_loop(0, NCH // 8, body, 0)
    for jj in range(NCH - 4, NCH):
        b = jj % 8
        pltpu.make_async_copy(rows_v.at[b], acc_sh.at[dst_v.at[jj]], sem_s[b]).wait()
    plsc.subcore_barrier()

    @pl.when(sid < NS - 1)
    def _sfull():
        pltpu.sync_copy(acc_sh.at[pl.ds(sid * RPT, RPT)], s_v)
    @pl.when(sid == NS - 1)
    def _stail():
        pltpu.sync_copy(acc_sh.at[pl.ds((NS - 1) * RPT, NLAST)],
                        s_v.at[pl.ds(0, NLAST)])
    w = jnp.where(cid == 0, 1.0, 0.0).astype(jnp.float32)
    def obody(i, carry):
        sp = plsc.load_gather(dis_v, [jnp.full((C,), i, jnp.int32)])
        s_v[i, :] = (s_v[i, :] + w * stage_v[i, :]) * sp
        return carry
    lax.fori_loop(0, RPT, obody, 0)
    @pl.when(sid < NS - 1)
    def _ofull():
        pltpu.sync_copy(s_v, out_hbm.at[cid, pl.ds(sid * RPT, RPT)])
    @pl.when(sid == NS - 1)
    def _otail():
        pltpu.sync_copy(s_v.at[pl.ds(0, NLAST)],
                        out_hbm.at[cid, pl.ds((NS - 1) * RPT, NLAST)])


def _matmul_h_body(x_ref, w2_ref, h_ref):
    h_ref[...] = jnp.dot(x_ref[...], w2_ref[...], preferred_element_type=jnp.float32)


_matmul_h = pl.pallas_call(
    _matmul_h_body,
    grid=(GRID,),
    in_specs=[
        pl.BlockSpec((BLK, F), lambda j: (j, 0)),
        pl.BlockSpec((F, C), lambda j: (0, 0)),
    ],
    out_specs=pl.BlockSpec((BLK, C), lambda j: (j, 0)),
    out_shape=jax.ShapeDtypeStruct((N, C), jnp.float32),
)


def _dense_out_body(z_ref, b2_ref, wl_ref, bl_ref, o_ref):
    t = jnp.maximum(z_ref[0] + z_ref[1] + b2_ref[...], 0.0)
    o_ref[...] = jnp.dot(t, wl_ref[...], preferred_element_type=jnp.float32) + bl_ref[...]


_dense_out = pl.pallas_call(
    _dense_out_body,
    out_shape=jax.ShapeDtypeStruct((N * C // 128, 128), jnp.float32),
)


def kernel(x, edge_index, W2, b2, Wl, bl):
    e_r = edge_index.astype(jnp.int32).reshape(2, NW, NCH, CHUNK)
    deg2 = _degree_kernel(e_r, jnp.zeros((160,), jnp.float32),
                          jnp.ones((CHUNK,), jnp.float32))
    h = _matmul_h(x, W2)
    z2 = _propagate_kernel(e_r, h, deg2, jnp.zeros((RPT, C), jnp.float32))
    z2p = z2.reshape(NC, N * C // 128, 128)
    wlk = jnp.kron(jnp.eye(8, dtype=jnp.float32), Wl)
    b2p = jnp.tile(b2, 8).reshape(1, 128)
    blp = jnp.tile(bl, 8).reshape(1, 128)
    outw = _dense_out(z2p, b2p, wlk, blp)
    return outw.reshape(N, C)


# speedup vs baseline: 92.3824x; 1.0090x over previous
import functools

import jax
import jax.numpy as jnp
from jax import lax
from jax.experimental import pallas as pl
from jax.experimental.pallas import tpu as pltpu
from jax.experimental.pallas import tpu_sc as plsc

N = 10000
F = 128
C = 16
NC = 2
NS = 16
NW = NC * NS
CHUNK = 125
NCH = 80
NPAD = 10240
RPT = NPAD // NS
NLAST = N - (NS - 1) * RPT
BLK = 2000
GRID = N // BLK

_mesh = plsc.VectorSubcoreMesh(core_axis_name="c", subcore_axis_name="s")
_sc_params = pltpu.CompilerParams(use_tc_tiling_on_sc=False,
                                  needs_layout_passes=False)


@functools.partial(
    pl.kernel,
    mesh=_mesh,
    out_type=jax.ShapeDtypeStruct((NC, NPAD), jnp.float32),
    compiler_params=_sc_params,
    scratch_types=[
        pltpu.VMEM((NCH, CHUNK), jnp.int32),
        pltpu.VMEM((160,), jnp.float32),
        pltpu.VMEM((CHUNK,), jnp.float32),
        pltpu.VMEM_SHARED((NPAD,), jnp.float32),
    ] + [pltpu.SemaphoreType.DMA] * 8,
)
def _degree_kernel(e_hbm, z_hbm, o_hbm, out_hbm, dst_v, zbuf_v, ones_v, acc_sh, *sems):
    cid = lax.axis_index("c")
    sid = lax.axis_index("s")
    wid = cid * NS + sid
    pltpu.sync_copy(z_hbm, zbuf_v)
    for k in range(RPT // 160):
        pltpu.sync_copy(zbuf_v, acc_sh.at[pl.ds(sid * RPT + k * 160, 160)])
    pltpu.sync_copy(o_hbm, ones_v)
    pltpu.sync_copy(e_hbm.at[1, wid], dst_v)
    plsc.subcore_barrier()

    def body(i, carry):
        for b in range(8):
            j = i * 8 + b
            @pl.when(j >= 8)
            def _drain():
                pltpu.make_async_copy(ones_v, acc_sh.at[dst_v.at[j - 8]], sems[b]).wait()
            pltpu.async_copy(ones_v, acc_sh.at[dst_v.at[j]], sems[b], add=True)
        return carry

    lax.fori_loop(0, NCH // 8, body, 0)
    for jj in range(NCH - 8, NCH):
        pltpu.make_async_copy(ones_v, acc_sh.at[dst_v.at[jj]], sems[jj % 8]).wait()
    plsc.subcore_barrier()
    pltpu.sync_copy(acc_sh.at[pl.ds(sid * RPT, RPT)],
                    out_hbm.at[cid, pl.ds(sid * RPT, RPT)])


def _rsqrt_newton(d):
    i = plsc.bitcast(d, jnp.int32)
    i = jnp.int32(0x5F3759DF) - lax.shift_right_arithmetic(i, 1)
    y = plsc.bitcast(i, jnp.float32)
    for _ in range(3):
        y = y * (1.5 - 0.5 * d * y * y)
    return y


@functools.partial(
    pl.kernel,
    mesh=_mesh,
    out_type=jax.ShapeDtypeStruct((NC, N, C), jnp.float32),
    compiler_params=_sc_params,
    scratch_types=[
        pltpu.VMEM((NCH, CHUNK), jnp.int32),
        pltpu.VMEM((NCH, CHUNK), jnp.int32),
        pltpu.VMEM((8, CHUNK, C), jnp.float32),
        pltpu.VMEM((RPT, C), jnp.float32),
        pltpu.VMEM((RPT, C), jnp.float32),
        pltpu.VMEM((RPT,), jnp.float32),
        pltpu.VMEM((RPT,), jnp.float32),
        pltpu.VMEM((RPT,), jnp.float32),
        pltpu.VMEM_SHARED((N, C), jnp.float32),
        pltpu.VMEM_SHARED((N, C), jnp.float32),
    ] + [pltpu.SemaphoreType.DMA] * 16,
)
def _propagate_kernel(e_hbm, h_hbm, deg_hbm, z_hbm, out_hbm,
                      src_v, dst_v, rows_v, stage_v, s_v,
                      deg0_v, deg1_v, dis_v, g_sh, acc_sh, *sems):
    cid = lax.axis_index("c")
    sid = lax.axis_index("s")
    wid = cid * NS + sid
    sem_g = sems[:8]
    sem_s = sems[8:]

    @pl.when(sid < NS - 1)
    def _full():
        pltpu.sync_copy(h_hbm.at[pl.ds(sid * RPT, RPT)], stage_v)
    @pl.when(sid == NS - 1)
    def _tail():
        pltpu.sync_copy(h_hbm.at[pl.ds((NS - 1) * RPT, NLAST)],
                        stage_v.at[pl.ds(0, NLAST)])
    pltpu.sync_copy(deg_hbm.at[0, pl.ds(sid * RPT, RPT)], deg0_v)
    pltpu.sync_copy(deg_hbm.at[1, pl.ds(sid * RPT, RPT)], deg1_v)

    @pl.when(sid < NS - 1)
    def _zfull():
        pltpu.sync_copy(z_hbm, acc_sh.at[pl.ds(sid * RPT, RPT)])
    @pl.when(sid == NS - 1)
    def _ztail():
        pltpu.sync_copy(z_hbm.at[pl.ds(0, NLAST)],
                        acc_sh.at[pl.ds((NS - 1) * RPT, NLAST)])

    def dbody(k, carry):
        d = deg0_v[pl.ds(k * C, C)] + deg1_v[pl.ds(k * C, C)] + 1.0
        dis_v[pl.ds(k * C, C)] = _rsqrt_newton(d)
        return carry
    lax.fori_loop(0, RPT // C, dbody, 0)

    def gbody(k, carry):
        for u in range(4):
            i = k * 4 + u
            sp = plsc.load_gather(dis_v, [jnp.full((C,), i, jnp.int32)])
            stage_v[i, :] = stage_v[i, :] * sp
        return carry
    lax.fori_loop(0, RPT // 4, gbody, 0)
    @pl.when(sid < NS - 1)
    def _gfull():
        pltpu.sync_copy(stage_v, g_sh.at[pl.ds(sid * RPT, RPT)])
    @pl.when(sid == NS - 1)
    def _gtail():
        pltpu.sync_copy(stage_v.at[pl.ds(0, NLAST)],
                        g_sh.at[pl.ds((NS - 1) * RPT, NLAST)])

    pltpu.sync_copy(e_hbm.at[0, wid], src_v)
    pltpu.sync_copy(e_hbm.at[1, wid], dst_v)
    plsc.subcore_barrier()

    for b in range(4):
        pltpu.async_copy(g_sh.at[src_v.at[b]], rows_v.at[b], sem_g[b])

    def body(i, carry):
        for b in range(8):
            j = i * 8 + b
            bp = (b + 4) % 8
            pltpu.make_async_copy(g_sh.at[src_v.at[j]], rows_v.at[b], sem_g[b]).wait()
            pltpu.async_copy(rows_v.at[b], acc_sh.at[dst_v.at[j]], sem_s[b], add=True)
            @pl.when(j >= 4)
            def _drain():
                pltpu.make_async_copy(rows_v.at[bp], acc_sh.at[dst_v.at[j - 4]], sem_s[bp]).wait()
            @pl.when(j < NCH - 4)
            def _prefetch():
                pltpu.async_copy(g_sh.at[src_v.at[j + 4]], rows_v.at[bp], sem_g[bp])
        return carry

    lax.fori_loop(0, NCH // 8, body, 0)
    for jj in range(NCH - 4, NCH):
        b = jj % 8
        pltpu.make_async_copy(rows_v.at[b], acc_sh.at[dst_v.at[jj]], sem_s[b]).wait()
    plsc.subcore_barrier()

    @pl.when(sid < NS - 1)
    def _sfull():
        pltpu.sync_copy(acc_sh.at[pl.ds(sid * RPT, RPT)], s_v)
    @pl.when(sid == NS - 1)
    def _stail():
        pltpu.sync_copy(acc_sh.at[pl.ds((NS - 1) * RPT, NLAST)],
                        s_v.at[pl.ds(0, NLAST)])
    w = jnp.where(cid == 0, 1.0, 0.0).astype(jnp.float32)
    def obody(k, carry):
        for u in range(4):
            i = k * 4 + u
            sp = plsc.load_gather(dis_v, [jnp.full((C,), i, jnp.int32)])
            s_v[i, :] = (s_v[i, :] + w * stage_v[i, :]) * sp
        return carry
    lax.fori_loop(0, RPT // 4, obody, 0)
    @pl.when(sid < NS - 1)
    def _ofull():
        pltpu.sync_copy(s_v, out_hbm.at[cid, pl.ds(sid * RPT, RPT)])
    @pl.when(sid == NS - 1)
    def _otail():
        pltpu.sync_copy(s_v.at[pl.ds(0, NLAST)],
                        out_hbm.at[cid, pl.ds((NS - 1) * RPT, NLAST)])


def _matmul_h_body(x_ref, w2_ref, h_ref):
    h_ref[...] = jnp.dot(x_ref[...], w2_ref[...], preferred_element_type=jnp.float32)


_matmul_h = pl.pallas_call(
    _matmul_h_body,
    grid=(GRID,),
    in_specs=[
        pl.BlockSpec((BLK, F), lambda j: (j, 0)),
        pl.BlockSpec((F, C), lambda j: (0, 0)),
    ],
    out_specs=pl.BlockSpec((BLK, C), lambda j: (j, 0)),
    out_shape=jax.ShapeDtypeStruct((N, C), jnp.float32),
)


def _dense_out_body(z_ref, b2_ref, wl_ref, bl_ref, o_ref):
    t = jnp.maximum(z_ref[0] + z_ref[1] + b2_ref[...], 0.0)
    o_ref[...] = jnp.dot(t, wl_ref[...], preferred_element_type=jnp.float32) + bl_ref[...]


_dense_out = pl.pallas_call(
    _dense_out_body,
    out_shape=jax.ShapeDtypeStruct((N * C // 128, 128), jnp.float32),
)


def kernel(x, edge_index, W2, b2, Wl, bl):
    e_r = edge_index.astype(jnp.int32).reshape(2, NW, NCH, CHUNK)
    deg2 = _degree_kernel(e_r, jnp.zeros((160,), jnp.float32),
                          jnp.ones((CHUNK,), jnp.float32))
    h = _matmul_h(x, W2)
    z2 = _propagate_kernel(e_r, h, deg2, jnp.zeros((RPT, C), jnp.float32))
    z2p = z2.reshape(NC, N * C // 128, 128)
    wlk = jnp.kron(jnp.eye(8, dtype=jnp.float32), Wl)
    b2p = jnp.tile(b2, 8).reshape(1, 128)
    blp = jnp.tile(bl, 8).reshape(1, 128)
    outw = _dense_out(z2p, b2p, wlk, blp)
    return outw.reshape(N, C)


# speedup vs baseline: 98.5748x; 1.0670x over previous
import functools

import jax
import jax.numpy as jnp
from jax import lax
from jax.experimental import pallas as pl
from jax.experimental.pallas import tpu as pltpu
from jax.experimental.pallas import tpu_sc as plsc

N = 10000
F = 128
C = 16
NC = 2
NS = 16
NW = NC * NS
EPT = 10000
CHUNK = 128
NCH = 78
TAIL = EPT - NCH * CHUNK
NPAD = 10240
RPT = NPAD // NS
NLAST = N - (NS - 1) * RPT
BLK = 2000
GRID = N // BLK

_mesh = plsc.VectorSubcoreMesh(core_axis_name="c", subcore_axis_name="s")
_sc_params = pltpu.CompilerParams(use_tc_tiling_on_sc=False,
                                  needs_layout_passes=False)


@functools.partial(
    pl.kernel,
    mesh=_mesh,
    out_type=jax.ShapeDtypeStruct((NC, NPAD), jnp.float32),
    compiler_params=_sc_params,
    scratch_types=[
        pltpu.VMEM((EPT,), jnp.int32),
        pltpu.VMEM((160,), jnp.float32),
        pltpu.VMEM((CHUNK,), jnp.float32),
        pltpu.VMEM_SHARED((NPAD,), jnp.float32),
    ] + [pltpu.SemaphoreType.DMA] * 8,
)
def _degree_kernel(e_hbm, z_hbm, o_hbm, out_hbm, dst_v, zbuf_v, ones_v, acc_sh, *sems):
    cid = lax.axis_index("c")
    sid = lax.axis_index("s")
    wid = cid * NS + sid
    pltpu.sync_copy(z_hbm, zbuf_v)
    for k in range(RPT // 160):
        pltpu.sync_copy(zbuf_v, acc_sh.at[pl.ds(sid * RPT + k * 160, 160)])
    pltpu.sync_copy(o_hbm, ones_v)
    pltpu.sync_copy(e_hbm.at[1, pl.ds(wid * EPT, EPT)], dst_v)
    plsc.subcore_barrier()

    def body(i, carry):
        for b in range(8):
            j = i * 8 + b
            @pl.when(j >= 8)
            def _drain():
                pltpu.make_async_copy(
                    ones_v, acc_sh.at[dst_v.at[pl.ds((j - 8) * CHUNK, CHUNK)]],
                    sems[b]).wait()
            pltpu.async_copy(
                ones_v, acc_sh.at[dst_v.at[pl.ds(j * CHUNK, CHUNK)]],
                sems[b], add=True)
        return carry

    lax.fori_loop(0, 9, body, 0)
    for jj in range(72, NCH):
        pltpu.make_async_copy(
            ones_v, acc_sh.at[dst_v.at[pl.ds((jj - 8) * CHUNK, CHUNK)]],
            sems[jj % 8]).wait()
        pltpu.async_copy(
            ones_v, acc_sh.at[dst_v.at[pl.ds(jj * CHUNK, CHUNK)]],
            sems[jj % 8], add=True)
    for jj in range(NCH - 8, NCH):
        pltpu.make_async_copy(
            ones_v, acc_sh.at[dst_v.at[pl.ds(jj * CHUNK, CHUNK)]],
            sems[jj % 8]).wait()
    pltpu.async_copy(
        ones_v.at[pl.ds(0, TAIL)],
        acc_sh.at[dst_v.at[pl.ds(NCH * CHUNK, TAIL)]], sems[0], add=True).wait()
    plsc.subcore_barrier()
    pltpu.sync_copy(acc_sh.at[pl.ds(sid * RPT, RPT)],
                    out_hbm.at[cid, pl.ds(sid * RPT, RPT)])


def _rsqrt_newton(d):
    i = plsc.bitcast(d, jnp.int32)
    i = jnp.int32(0x5F3759DF) - lax.shift_right_arithmetic(i, 1)
    y = plsc.bitcast(i, jnp.float32)
    for _ in range(3):
        y = y * (1.5 - 0.5 * d * y * y)
    return y


@functools.partial(
    pl.kernel,
    mesh=_mesh,
    out_type=jax.ShapeDtypeStruct((NC, N, C), jnp.float32),
    compiler_params=_sc_params,
    scratch_types=[
        pltpu.VMEM((EPT,), jnp.int32),
        pltpu.VMEM((EPT,), jnp.int32),
        pltpu.VMEM((8, CHUNK, C), jnp.float32),
        pltpu.VMEM((RPT, C), jnp.float32),
        pltpu.VMEM((RPT, C), jnp.float32),
        pltpu.VMEM((RPT,), jnp.float32),
        pltpu.VMEM((RPT,), jnp.float32),
        pltpu.VMEM((RPT,), jnp.float32),
        pltpu.VMEM_SHARED((N, C), jnp.float32),
        pltpu.VMEM_SHARED((N, C), jnp.float32),
    ] + [pltpu.SemaphoreType.DMA] * 16,
)
def _propagate_kernel(e_hbm, h_hbm, deg_hbm, z_hbm, out_hbm,
                      src_v, dst_v, rows_v, stage_v, s_v,
                      deg0_v, deg1_v, dis_v, g_sh, acc_sh, *sems):
    cid = lax.axis_index("c")
    sid = lax.axis_index("s")
    wid = cid * NS + sid
    sem_g = sems[:8]
    sem_s = sems[8:]

    @pl.when(sid < NS - 1)
    def _full():
        pltpu.sync_copy(h_hbm.at[pl.ds(sid * RPT, RPT)], stage_v)
    @pl.when(sid == NS - 1)
    def _tail():
        pltpu.sync_copy(h_hbm.at[pl.ds((NS - 1) * RPT, NLAST)],
                        stage_v.at[pl.ds(0, NLAST)])
    pltpu.sync_copy(deg_hbm.at[0, pl.ds(sid * RPT, RPT)], deg0_v)
    pltpu.sync_copy(deg_hbm.at[1, pl.ds(sid * RPT, RPT)], deg1_v)

    @pl.when(sid < NS - 1)
    def _zfull():
        pltpu.sync_copy(z_hbm, acc_sh.at[pl.ds(sid * RPT, RPT)])
    @pl.when(sid == NS - 1)
    def _ztail():
        pltpu.sync_copy(z_hbm.at[pl.ds(0, NLAST)],
                        acc_sh.at[pl.ds((NS - 1) * RPT, NLAST)])

    def dbody(k, carry):
        d = deg0_v[pl.ds(k * C, C)] + deg1_v[pl.ds(k * C, C)] + 1.0
        dis_v[pl.ds(k * C, C)] = _rsqrt_newton(d)
        return carry
    lax.fori_loop(0, RPT // C, dbody, 0)

    def gbody(k, carry):
        for u in range(4):
            i = k * 4 + u
            sp = plsc.load_gather(dis_v, [jnp.full((C,), i, jnp.int32)])
            stage_v[i, :] = stage_v[i, :] * sp
        return carry
    lax.fori_loop(0, RPT // 4, gbody, 0)
    @pl.when(sid < NS - 1)
    def _gfull():
        pltpu.sync_copy(stage_v, g_sh.at[pl.ds(sid * RPT, RPT)])
    @pl.when(sid == NS - 1)
    def _gtail():
        pltpu.sync_copy(stage_v.at[pl.ds(0, NLAST)],
                        g_sh.at[pl.ds((NS - 1) * RPT, NLAST)])

    pltpu.sync_copy(e_hbm.at[0, pl.ds(wid * EPT, EPT)], src_v)
    pltpu.sync_copy(e_hbm.at[1, pl.ds(wid * EPT, EPT)], dst_v)
    plsc.subcore_barrier()

    def sidx(j):
        return src_v.at[pl.ds(j * CHUNK, CHUNK)]
    def didx(j):
        return dst_v.at[pl.ds(j * CHUNK, CHUNK)]

    for b in range(4):
        pltpu.async_copy(g_sh.at[sidx(b)], rows_v.at[b], sem_g[b])

    def body(i, carry):
        for b in range(8):
            j = i * 8 + b
            bp = (b + 4) % 8
            pltpu.make_async_copy(g_sh.at[sidx(j)], rows_v.at[b], sem_g[b]).wait()
            pltpu.async_copy(rows_v.at[b], acc_sh.at[didx(j)], sem_s[b], add=True)
            @pl.when(j >= 4)
            def _drain():
                pltpu.make_async_copy(rows_v.at[bp], acc_sh.at[didx(j - 4)], sem_s[bp]).wait()
            pltpu.async_copy(g_sh.at[sidx(j + 4)], rows_v.at[bp], sem_g[bp])
        return carry

    lax.fori_loop(0, 9, body, 0)
    for jj in range(72, NCH):
        b = jj % 8
        bp = (b + 4) % 8
        pltpu.make_async_copy(g_sh.at[sidx(jj)], rows_v.at[b], sem_g[b]).wait()
        pltpu.async_copy(rows_v.at[b], acc_sh.at[didx(jj)], sem_s[b], add=True)
        pltpu.make_async_copy(rows_v.at[bp], acc_sh.at[didx(jj - 4)], sem_s[bp]).wait()
        if jj + 4 < NCH:
            pltpu.async_copy(g_sh.at[sidx(jj + 4)], rows_v.at[bp], sem_g[bp])
    for jj in range(NCH - 4, NCH):
        b = jj % 8
        pltpu.make_async_copy(rows_v.at[b], acc_sh.at[didx(jj)], sem_s[b]).wait()
    pltpu.async_copy(g_sh.at[src_v.at[pl.ds(NCH * CHUNK, TAIL)]],
                     rows_v.at[0, pl.ds(0, TAIL)], sem_g[0]).wait()
    pltpu.async_copy(rows_v.at[0, pl.ds(0, TAIL)],
                     acc_sh.at[dst_v.at[pl.ds(NCH * CHUNK, TAIL)]],
                     sem_s[0], add=True).wait()
    plsc.subcore_barrier()

    @pl.when(sid < NS - 1)
    def _sfull():
        pltpu.sync_copy(acc_sh.at[pl.ds(sid * RPT, RPT)], s_v)
    @pl.when(sid == NS - 1)
    def _stail():
        pltpu.sync_copy(acc_sh.at[pl.ds((NS - 1) * RPT, NLAST)],
                        s_v.at[pl.ds(0, NLAST)])
    w = jnp.where(cid == 0, 1.0, 0.0).astype(jnp.float32)
    def obody(k, carry):
        for u in range(4):
            i = k * 4 + u
            sp = plsc.load_gather(dis_v, [jnp.full((C,), i, jnp.int32)])
            s_v[i, :] = (s_v[i, :] + w * stage_v[i, :]) * sp
        return carry
    lax.fori_loop(0, RPT // 4, obody, 0)
    @pl.when(sid < NS - 1)
    def _ofull():
        pltpu.sync_copy(s_v, out_hbm.at[cid, pl.ds(sid * RPT, RPT)])
    @pl.when(sid == NS - 1)
    def _otail():
        pltpu.sync_copy(s_v.at[pl.ds(0, NLAST)],
                        out_hbm.at[cid, pl.ds((NS - 1) * RPT, NLAST)])


def _matmul_h_body(x_ref, w2_ref, h_ref):
    h_ref[...] = jnp.dot(x_ref[...], w2_ref[...], preferred_element_type=jnp.float32)


_matmul_h = pl.pallas_call(
    _matmul_h_body,
    grid=(GRID,),
    in_specs=[
        pl.BlockSpec((BLK, F), lambda j: (j, 0)),
        pl.BlockSpec((F, C), lambda j: (0, 0)),
    ],
    out_specs=pl.BlockSpec((BLK, C), lambda j: (j, 0)),
    out_shape=jax.ShapeDtypeStruct((N, C), jnp.float32),
)


def _dense_out_body(z_ref, b2_ref, wl_ref, bl_ref, o_ref):
    t = jnp.maximum(z_ref[0] + z_ref[1] + b2_ref[...], 0.0)
    o_ref[...] = jnp.dot(t, wl_ref[...], preferred_element_type=jnp.float32) + bl_ref[...]


_dense_out = pl.pallas_call(
    _dense_out_body,
    out_shape=jax.ShapeDtypeStruct((N * C // 128, 128), jnp.float32),
)


def kernel(x, edge_index, W2, b2, Wl, bl):
    e32 = edge_index.astype(jnp.int32)
    deg2 = _degree_kernel(e32, jnp.zeros((160,), jnp.float32),
                          jnp.ones((CHUNK,), jnp.float32))
    h = _matmul_h(x, W2)
    z2 = _propagate_kernel(e32, h, deg2, jnp.zeros((RPT, C), jnp.float32))
    z2p = z2.reshape(NC, N * C // 128, 128)
    wlk = jnp.kron(jnp.eye(8, dtype=jnp.float32), Wl)
    b2p = jnp.tile(b2, 8).reshape(1, 128)
    blp = jnp.tile(bl, 8).reshape(1, 128)
    outw = _dense_out(z2p, b2p, wlk, blp)
    return outw.reshape(N, C)
